# baseline (speedup 1.0000x reference)
import functools

import jax
import jax.numpy as jnp
from jax import lax
from jax.experimental import pallas as pl
from jax.experimental.pallas import tpu as pltpu
from jax.experimental.pallas import tpu_sc as plsc


_STREAM_ONLY = False


def kernel(index, train_score):
    (B,) = index.shape
    V, D = train_score.shape
    L = 16

    info = plsc.get_sparse_core_info()
    nc, ns = info.num_cores, info.num_subcores
    nw = nc * ns
    CT = V // 128
    TAIL = V - CT * 128
    ct_per_w = CT // nw
    WT = 10
    W = WT * 128
    NBUF = 2
    ct_last = CT - (nw - 1) * ct_per_w
    nwin = -(-ct_last // WT)
    n_filter = B // L
    OPAD = 128
    EC = 64
    mesh = plsc.VectorSubcoreMesh(core_axis_name="c", subcore_axis_name="s")

    @functools.partial(
        pl.kernel,
        mesh=mesh,
        out_type=jax.ShapeDtypeStruct((B, OPAD), jnp.float32),
        compiler_params=pltpu.CompilerParams(needs_layout_passes=False),
        scratch_types=[
            pltpu.VMEM((B,), jnp.int32),
            pltpu.VMEM((B + L,), jnp.int32),
            pltpu.VMEM((B + L,), jnp.int32),
            pltpu.VMEM((NBUF, D, W), jnp.float32),
            pltpu.VMEM((EC, OPAD), jnp.float32),
            pltpu.VMEM((EC,), jnp.int32),
            pltpu.VMEM((D, max(TAIL, 1)), jnp.float32),
        ] + [pltpu.SemaphoreType.DMA] * (NBUF + 1),
    )
    def scan_gather(idx_hbm, table_t_hbm, out_hbm, idx_v, bin_pos, wpos,
                    win_ring, stage, sidx_v, tail_buf, *sems_all):
        sems, sem_i = sems_all[:NBUF], sems_all[NBUF]
        wid = lax.axis_index("s") * nc + lax.axis_index("c")
        ct0 = wid * ct_per_w
        v0 = ct0 * 128
        v1 = jnp.where(wid == nw - 1, V, (ct0 + ct_per_w) * 128)
        iota = lax.iota(jnp.int32, L)
        wins = tuple(win_ring.at[b] for b in range(NBUF))

        def win_start(w):
            ws = jnp.minimum(ct0 + w * WT, CT - WT) * 128
            return pl.multiple_of(ws, 128)

        def start_window(w, buf, sem):
            ws = win_start(w)
            return [
                pltpu.async_copy(
                    table_t_hbm.at[h, :, pl.ds(ws, W)],
                    buf.at[pl.ds(8 * h, 8), :], sem)
                for h in range(2)
            ]

        def drain_window(buf, sem):
            for h in range(2):
                pltpu.make_async_copy(
                    table_t_hbm.at[h, :, pl.ds(0, W)],
                    buf.at[pl.ds(8 * h, 8), :], sem).wait()

        def process(win, wv0, width, wcap):
            if _STREAM_ONLY:
                return
            def sfbody(j, c):
                valid = (j * L + iota) < bin_cnt
                pos = bin_pos[pl.ds(j * L, L)]
                pos = jnp.where(valid, pos, 0)
                vv = plsc.load_gather(idx_v, [pos])
                m = (valid & (vv >= wv0) & (vv < wv0 + width)
                     & (vv >= v0) & (vv < v1))
                slot = c + plsc.cumsum(m.astype(jnp.int32)) - 1
                plsc.store_scatter(wpos, [slot], pos, mask=m)
                return c + jnp.sum(m.astype(jnp.int32))

            wcnt = lax.fori_loop(0, n_chunks, sfbody, 0, unroll=False)
            n_ech = lax.div(wcnt + (EC - 1), EC)

            def ebody(t, _):
                base = t * EC
                for q in range(EC // L):
                    valid = (base + q * L + iota) < wcnt
                    pos = wpos[pl.ds(base + q * L, L)]
                    vv = plsc.load_gather(idx_v, [jnp.where(valid, pos, 0)])
                    loc = jnp.where(valid, vv - wv0, 0)
                    for d in range(D):
                        vals = plsc.load_gather(
                            win, [jnp.full((L,), d, jnp.int32), loc])
                        plsc.store_scatter(
                            stage,
                            [q * L + iota, jnp.full((L,), d, jnp.int32)], vals)
                    sidx_v[pl.ds(q * L, L)] = jnp.where(valid, pos, -1)
                pltpu.sync_copy(
                    stage,
                    out_hbm.at[plsc.Indices(sidx_v, ignored_value=-1)])
                return 0

            lax.fori_loop(0, n_ech, ebody, 0, unroll=False)
            del wcap

        start_window(0, wins[0], sems[0])
        pltpu.async_copy(idx_hbm, idx_v, sem_i).wait()

        def fbody(k, cnt):
            vv = idx_v[pl.ds(k * L, L)]
            m = (vv >= v0) & (vv < v1)
            slot = cnt + plsc.cumsum(m.astype(jnp.int32)) - 1
            plsc.store_scatter(bin_pos, [slot], k * L + iota, mask=m)
            return cnt + jnp.sum(m.astype(jnp.int32))

        bin_cnt = lax.fori_loop(0, n_filter, fbody, 0, unroll=False)
        n_chunks = lax.div(bin_cnt + (L - 1), L)

        def wbody(w, _):
            @pl.when(w + 1 < nwin)
            def _prefetch():
                @pl.when(lax.rem(w + 1, 2) == 0)
                def _():
                    start_window(w + 1, wins[0], sems[0])

                @pl.when(lax.rem(w + 1, 2) == 1)
                def _():
                    start_window(w + 1, wins[1], sems[1])

            @pl.when(lax.rem(w, 2) == 0)
            def _even():
                drain_window(wins[0], sems[0])
                process(wins[0], win_start(w), W, W)

            @pl.when(lax.rem(w, 2) == 1)
            def _odd():
                drain_window(wins[1], sems[1])
                process(wins[1], win_start(w), W, W)

            return 0

        lax.fori_loop(0, nwin, wbody, 0, unroll=False)

        if TAIL:
            @pl.when(wid == nw - 1)
            def _():
                tail_copies = [
                    pltpu.async_copy(
                        table_t_hbm.at[h, :, pl.ds(CT * 128, TAIL)],
                        tail_buf.at[pl.ds(8 * h, 8), :], sem_i)
                    for h in range(2)
                ]
                for c in tail_copies:
                    c.wait()
                process(tail_buf, CT * 128, TAIL, TAIL)

    out_pad = scan_gather(index, train_score.T.reshape(2, D // 2, V))
    return out_pad[:, :D]


# speedup vs baseline: 1.0469x; 1.0469x over previous
import functools

import jax
import jax.numpy as jnp
from jax import lax
from jax.experimental import pallas as pl
from jax.experimental.pallas import tpu as pltpu
from jax.experimental.pallas import tpu_sc as plsc


_STREAM_ONLY = False


def kernel(index, train_score):
    (B,) = index.shape
    V, D = train_score.shape
    L = 16

    info = plsc.get_sparse_core_info()
    nc, ns = info.num_cores, info.num_subcores
    nw = nc * ns
    CT = V // 128
    TAIL = V - CT * 128
    ct_per_w = CT // nw
    WT = 10
    W = WT * 128
    NBUF = 3
    ct_last = CT - (nw - 1) * ct_per_w
    nwin = -(-ct_last // WT)
    n_filter = B // L
    OPAD = 128
    EC = 16
    mesh = plsc.VectorSubcoreMesh(core_axis_name="c", subcore_axis_name="s")

    @functools.partial(
        pl.kernel,
        mesh=mesh,
        out_type=jax.ShapeDtypeStruct((B, OPAD), jnp.float32),
        compiler_params=pltpu.CompilerParams(needs_layout_passes=False),
        scratch_types=[
            pltpu.VMEM((B,), jnp.int32),
            pltpu.VMEM((B + L,), jnp.int32),
            pltpu.VMEM((B + L,), jnp.int32),
            pltpu.VMEM((NBUF, D, W), jnp.float32),
            pltpu.VMEM((EC, OPAD), jnp.float32),
            pltpu.VMEM((EC,), jnp.int32),
            pltpu.VMEM((D, max(TAIL, 1)), jnp.float32),
        ] + [pltpu.SemaphoreType.DMA] * (NBUF + 1),
    )
    def scan_gather(idx_hbm, table_t_hbm, out_hbm, idx_v, bin_pos, wpos,
                    win_ring, stage, sidx_v, tail_buf, *sems_all):
        sems, sem_i = sems_all[:NBUF], sems_all[NBUF]
        wid = lax.axis_index("s") * nc + lax.axis_index("c")
        ct0 = wid * ct_per_w
        v0 = ct0 * 128
        v1 = jnp.where(wid == nw - 1, V, (ct0 + ct_per_w) * 128)
        iota = lax.iota(jnp.int32, L)
        wins = tuple(win_ring.at[b] for b in range(NBUF))

        def win_start(w):
            ws = jnp.minimum(ct0 + w * WT, CT - WT) * 128
            return pl.multiple_of(ws, 128)

        def start_window(w, buf, sem):
            ws = win_start(w)
            return [
                pltpu.async_copy(
                    table_t_hbm.at[h, :, pl.ds(ws, W)],
                    buf.at[pl.ds(8 * h, 8), :], sem)
                for h in range(2)
            ]


        def process(win, wv0, width, wcap):
            if _STREAM_ONLY:
                return
            def sfbody(j, c):
                valid = (j * L + iota) < bin_cnt
                pos = bin_pos[pl.ds(j * L, L)]
                pos = jnp.where(valid, pos, 0)
                vv = plsc.load_gather(idx_v, [pos])
                m = (valid & (vv >= wv0) & (vv < wv0 + width)
                     & (vv >= v0) & (vv < v1))
                slot = c + plsc.cumsum(m.astype(jnp.int32)) - 1
                plsc.store_scatter(wpos, [slot], pos, mask=m)
                return c + jnp.sum(m.astype(jnp.int32))

            wcnt = lax.fori_loop(0, n_chunks, sfbody, 0, unroll=False)
            n_ech = lax.div(wcnt + (EC - 1), EC)

            def ebody(t, _):
                base = t * EC
                for q in range(EC // L):
                    valid = (base + q * L + iota) < wcnt
                    pos = wpos[pl.ds(base + q * L, L)]
                    vv = plsc.load_gather(idx_v, [jnp.where(valid, pos, 0)])
                    loc = jnp.where(valid, vv - wv0, 0)
                    for d in range(D):
                        vals = plsc.load_gather(
                            win, [jnp.full((L,), d, jnp.int32), loc])
                        plsc.store_scatter(
                            stage,
                            [q * L + iota, jnp.full((L,), d, jnp.int32)], vals)
                    sidx_v[pl.ds(q * L, L)] = jnp.where(valid, pos, -1)
                pltpu.sync_copy(
                    stage,
                    out_hbm.at[plsc.Indices(sidx_v, ignored_value=-1)])
                return 0

            lax.fori_loop(0, n_ech, ebody, 0, unroll=False)
            del wcap

        copies = {}
        for b in range(min(NBUF - 1, nwin)):
            copies[b] = start_window(b, wins[b], sems[b])
        pltpu.async_copy(idx_hbm, idx_v, sem_i).wait()

        def fbody(k, cnt):
            vv = idx_v[pl.ds(k * L, L)]
            m = (vv >= v0) & (vv < v1)
            slot = cnt + plsc.cumsum(m.astype(jnp.int32)) - 1
            plsc.store_scatter(bin_pos, [slot], k * L + iota, mask=m)
            return cnt + jnp.sum(m.astype(jnp.int32))

        bin_cnt = lax.fori_loop(0, n_filter, fbody, 0, unroll=False)
        n_chunks = lax.div(bin_cnt + (L - 1), L)

        for w in range(nwin):
            wp = w + NBUF - 1
            if wp < nwin:
                copies[wp % NBUF] = start_window(
                    wp, wins[wp % NBUF], sems[wp % NBUF])
            for c in copies[w % NBUF]:
                c.wait()
            process(wins[w % NBUF], win_start(w), W, W)

        if TAIL:
            @pl.when(wid == nw - 1)
            def _():
                tail_copies = [
                    pltpu.async_copy(
                        table_t_hbm.at[h, :, pl.ds(CT * 128, TAIL)],
                        tail_buf.at[pl.ds(8 * h, 8), :], sem_i)
                    for h in range(2)
                ]
                for c in tail_copies:
                    c.wait()
                process(tail_buf, CT * 128, TAIL, TAIL)

    out_pad = scan_gather(index, train_score.T.reshape(2, D // 2, V))
    return out_pad[:, :D]


# speedup vs baseline: 1.1846x; 1.1315x over previous
import functools

import jax
import jax.numpy as jnp
from jax import lax
from jax.experimental import pallas as pl
from jax.experimental.pallas import tpu as pltpu
from jax.experimental.pallas import tpu_sc as plsc


_STREAM_ONLY = False


def kernel(index, train_score):
    (B,) = index.shape
    V, D = train_score.shape
    L = 16

    info = plsc.get_sparse_core_info()
    nc, ns = info.num_cores, info.num_subcores
    nw = nc * ns
    CT = V // 128
    TAIL = V - CT * 128
    ct_per_w = CT // nw
    WT = 18
    W = WT * 128
    NBUF = 2
    ct_last = CT - (nw - 1) * ct_per_w
    nwin = -(-ct_last // WT)
    n_filter = B // L
    OPAD = 128
    EC = 16
    mesh = plsc.VectorSubcoreMesh(core_axis_name="c", subcore_axis_name="s")

    @functools.partial(
        pl.kernel,
        mesh=mesh,
        out_type=jax.ShapeDtypeStruct((B, OPAD), jnp.float32),
        compiler_params=pltpu.CompilerParams(needs_layout_passes=False),
        scratch_types=[
            pltpu.VMEM((B,), jnp.int32),
            pltpu.VMEM((B + L,), jnp.int32),
            pltpu.VMEM((B + L,), jnp.int32),
            pltpu.VMEM((NBUF, D, W), jnp.float32),
            pltpu.VMEM((EC, OPAD), jnp.float32),
            pltpu.VMEM((EC,), jnp.int32),
            pltpu.VMEM((D, max(TAIL, 1)), jnp.float32),
        ] + [pltpu.SemaphoreType.DMA] * (NBUF + 1),
    )
    def scan_gather(idx_hbm, table_t_hbm, out_hbm, idx_v, bin_pos, wpos,
                    win_ring, stage, sidx_v, tail_buf, *sems_all):
        sems, sem_i = sems_all[:NBUF], sems_all[NBUF]
        wid = lax.axis_index("s") * nc + lax.axis_index("c")
        ct0 = wid * ct_per_w
        v0 = ct0 * 128
        v1 = jnp.where(wid == nw - 1, V, (ct0 + ct_per_w) * 128)
        iota = lax.iota(jnp.int32, L)
        wins = tuple(win_ring.at[b] for b in range(NBUF))

        def win_start(w):
            ws = jnp.minimum(ct0 + w * WT, CT - WT) * 128
            return pl.multiple_of(ws, 128)

        def start_window(w, buf, sem):
            ws = win_start(w)
            return [
                pltpu.async_copy(
                    table_t_hbm.at[h, :, pl.ds(ws, W)],
                    buf.at[pl.ds(8 * h, 8), :], sem)
                for h in range(2)
            ]


        def process(win, wv0, width, wcap):
            if _STREAM_ONLY:
                return
            def sfbody(j, c):
                valid = (j * L + iota) < bin_cnt
                pos = bin_pos[pl.ds(j * L, L)]
                pos = jnp.where(valid, pos, 0)
                vv = plsc.load_gather(idx_v, [pos])
                m = (valid & (vv >= wv0) & (vv < wv0 + width)
                     & (vv >= v0) & (vv < v1))
                slot = c + plsc.cumsum(m.astype(jnp.int32)) - 1
                plsc.store_scatter(wpos, [slot], pos, mask=m)
                return c + jnp.sum(m.astype(jnp.int32))

            wcnt = lax.fori_loop(0, n_chunks, sfbody, 0, unroll=False)
            n_ech = lax.div(wcnt + (EC - 1), EC)

            def ebody(t, _):
                base = t * EC
                for q in range(EC // L):
                    valid = (base + q * L + iota) < wcnt
                    pos = wpos[pl.ds(base + q * L, L)]
                    vv = plsc.load_gather(idx_v, [jnp.where(valid, pos, 0)])
                    loc = jnp.where(valid, vv - wv0, 0)
                    for d in range(D):
                        vals = plsc.load_gather(
                            win, [jnp.full((L,), d, jnp.int32), loc])
                        plsc.store_scatter(
                            stage,
                            [q * L + iota, jnp.full((L,), d, jnp.int32)], vals)
                    sidx_v[pl.ds(q * L, L)] = jnp.where(valid, pos, -1)
                pltpu.sync_copy(
                    stage,
                    out_hbm.at[plsc.Indices(sidx_v, ignored_value=-1)])
                return 0

            lax.fori_loop(0, n_ech, ebody, 0, unroll=False)
            del wcap

        copies = {}
        for b in range(min(NBUF - 1, nwin)):
            copies[b] = start_window(b, wins[b], sems[b])
        pltpu.async_copy(idx_hbm, idx_v, sem_i).wait()

        def fbody(k, cnt):
            vv = idx_v[pl.ds(k * L, L)]
            m = (vv >= v0) & (vv < v1)
            slot = cnt + plsc.cumsum(m.astype(jnp.int32)) - 1
            plsc.store_scatter(bin_pos, [slot], k * L + iota, mask=m)
            return cnt + jnp.sum(m.astype(jnp.int32))

        bin_cnt = lax.fori_loop(0, n_filter, fbody, 0, unroll=False)
        n_chunks = lax.div(bin_cnt + (L - 1), L)

        for w in range(nwin):
            wp = w + NBUF - 1
            if wp < nwin:
                copies[wp % NBUF] = start_window(
                    wp, wins[wp % NBUF], sems[wp % NBUF])
            for c in copies[w % NBUF]:
                c.wait()
            process(wins[w % NBUF], win_start(w), W, W)

        if TAIL:
            @pl.when(wid == nw - 1)
            def _():
                tail_copies = [
                    pltpu.async_copy(
                        table_t_hbm.at[h, :, pl.ds(CT * 128, TAIL)],
                        tail_buf.at[pl.ds(8 * h, 8), :], sem_i)
                    for h in range(2)
                ]
                for c in tail_copies:
                    c.wait()
                process(tail_buf, CT * 128, TAIL, TAIL)

    out_pad = scan_gather(index, train_score.T.reshape(2, D // 2, V))
    return out_pad[:, :D]
